# baseline (device time: 26110 ns/iter reference)
import jax
import jax.numpy as jnp
from jax import lax
from jax.experimental import pallas as pl
from jax.experimental.pallas import tpu as pltpu

N_DEV = 4


def kernel(x, w_mat):
    m_per, k = x.shape
    _, n_per = w_mat.shape
    half = m_per // 2

    def body(x_hbm, w_hbm, out_ref, x_ref, w_ref,
             from_left, from_right, opp_buf,
             send_sems, recv_sems, copy_sems):
        my = lax.axis_index("i")
        left = (my - 1) % N_DEV
        right = (my + 1) % N_DEV
        opp = (my + 2) % N_DEV

        top = (pl.ds(0, half), slice(None))
        bot = (pl.ds(half, half), slice(None))

        cp_x = pltpu.make_async_copy(x_hbm, x_ref, copy_sems.at[0])
        cp_w = pltpu.make_async_copy(w_hbm, w_ref, copy_sems.at[1])
        cp_x.start()
        cp_w.start()

        barrier_sem = pltpu.get_barrier_semaphore()
        for nbr in (left, right):
            pl.semaphore_signal(
                barrier_sem, inc=1,
                device_id=(nbr,), device_id_type=pl.DeviceIdType.MESH,
            )
        pl.semaphore_wait(barrier_sem, 2)
        cp_x.wait()

        def rcopy(src, dst, i, dev):
            return pltpu.make_async_remote_copy(
                src_ref=src, dst_ref=dst,
                send_sem=send_sems.at[i], recv_sem=recv_sems.at[i],
                device_id=(dev,), device_id_type=pl.DeviceIdType.MESH,
            )

        a0 = rcopy(x_ref.at[top], from_left.at[top], 0, right)
        a1 = rcopy(x_ref.at[bot], from_left.at[bot], 1, right)
        a2 = rcopy(x_ref.at[bot], from_right.at[bot], 2, left)
        a3 = rcopy(x_ref.at[top], from_right.at[top], 3, left)
        a0.start()
        a2.start()
        a1.start()
        a3.start()
        cp_w.wait()

        def half_block(src_ref, origin, row0):
            y = jnp.dot(src_ref[pl.ds(row0, half), :], w_ref[:, :],
                        preferred_element_type=jnp.float32)
            out_ref[pl.ds(origin * m_per + row0, half), :] = (
                y * jax.nn.sigmoid(y))

        half_block(x_ref, my, 0)
        half_block(x_ref, my, half)

        a0.wait_recv()
        fwd_r = rcopy(from_left.at[top], opp_buf.at[top], 4, right)
        fwd_r.start()
        half_block(from_left, left, 0)

        a2.wait_recv()
        fwd_l = rcopy(from_right.at[bot], opp_buf.at[bot], 5, left)
        fwd_l.start()
        half_block(from_right, right, half)

        a1.wait_recv()
        half_block(from_left, left, half)
        a3.wait_recv()
        half_block(from_right, right, 0)

        fwd_r.wait_recv()
        half_block(opp_buf, opp, 0)
        fwd_l.wait_recv()
        half_block(opp_buf, opp, half)

        for r in (a0, a1, a2, a3, fwd_r, fwd_l):
            r.wait_send()

    return pl.pallas_call(
        body,
        out_shape=jax.ShapeDtypeStruct((N_DEV * m_per, n_per), jnp.float32),
        in_specs=[
            pl.BlockSpec(memory_space=pl.ANY),
            pl.BlockSpec(memory_space=pl.ANY),
        ],
        out_specs=pl.BlockSpec(memory_space=pltpu.VMEM),
        scratch_shapes=[
            pltpu.VMEM((m_per, k), jnp.float32),
            pltpu.VMEM((k, n_per), jnp.float32),
            pltpu.VMEM((m_per, k), jnp.float32),
            pltpu.VMEM((m_per, k), jnp.float32),
            pltpu.VMEM((m_per, k), jnp.float32),
            pltpu.SemaphoreType.DMA((6,)),
            pltpu.SemaphoreType.DMA((6,)),
            pltpu.SemaphoreType.DMA((2,)),
        ],
        compiler_params=pltpu.CompilerParams(collective_id=0),
    )(x, w_mat)


# device time: 25479 ns/iter; 1.0248x vs baseline; 1.0248x over previous
import jax
import jax.numpy as jnp
from jax import lax
from jax.experimental import pallas as pl
from jax.experimental.pallas import tpu as pltpu

N_DEV = 4


def kernel(x, w_mat):
    m_per, k = x.shape
    _, n_per = w_mat.shape
    half = m_per // 2

    def body(x_hbm, w_hbm, out_hbm, x_ref, w_ref, out_vmem,
             from_left, from_right, opp_buf,
             send_sems, recv_sems, stage_sems, out_sems):
        my = lax.axis_index("i")
        left = (my - 1) % N_DEV
        right = (my + 1) % N_DEV
        opp = (my + 2) % N_DEV

        top = (pl.ds(0, half), slice(None))
        bot = (pl.ds(half, half), slice(None))

        cp_x = pltpu.make_async_copy(x_hbm, x_ref, stage_sems.at[0])
        cp_w = pltpu.make_async_copy(w_hbm, w_ref, stage_sems.at[1])
        cp_x.start()
        cp_w.start()

        barrier_sem = pltpu.get_barrier_semaphore()
        for nbr in (left, right):
            pl.semaphore_signal(
                barrier_sem, inc=1,
                device_id=(nbr,), device_id_type=pl.DeviceIdType.MESH,
            )
        pl.semaphore_wait(barrier_sem, 2)
        cp_x.wait()

        def rcopy(src, dst, i, dev):
            return pltpu.make_async_remote_copy(
                src_ref=src, dst_ref=dst,
                send_sem=send_sems.at[i], recv_sem=recv_sems.at[i],
                device_id=(dev,), device_id_type=pl.DeviceIdType.MESH,
            )

        a0 = rcopy(x_ref.at[top], from_left.at[top], 0, right)
        a1 = rcopy(x_ref.at[bot], from_left.at[bot], 1, right)
        a2 = rcopy(x_ref.at[bot], from_right.at[bot], 2, left)
        a3 = rcopy(x_ref.at[top], from_right.at[top], 3, left)
        a0.start()
        a2.start()
        a1.start()
        a3.start()
        cp_w.wait()

        out_copies = []

        def half_block(src_ref, origin, row0):
            y = jnp.dot(src_ref[pl.ds(row0, half), :], w_ref[:, :],
                        preferred_element_type=jnp.float32)
            rows = pl.ds(origin * m_per + row0, half)
            out_vmem[rows, :] = y * jax.nn.sigmoid(y)
            cp = pltpu.make_async_copy(
                out_vmem.at[rows, :], out_hbm.at[rows, :],
                out_sems.at[len(out_copies)])
            cp.start()
            out_copies.append(cp)

        half_block(x_ref, my, 0)
        half_block(x_ref, my, half)

        a0.wait_recv()
        fwd_r = rcopy(from_left.at[top], opp_buf.at[top], 4, right)
        fwd_r.start()
        half_block(from_left, left, 0)

        a2.wait_recv()
        fwd_l = rcopy(from_right.at[bot], opp_buf.at[bot], 5, left)
        fwd_l.start()
        half_block(from_right, right, half)

        a1.wait_recv()
        half_block(from_left, left, half)
        a3.wait_recv()
        half_block(from_right, right, 0)

        fwd_r.wait_recv()
        half_block(opp_buf, opp, 0)
        fwd_l.wait_recv()
        half_block(opp_buf, opp, half)

        for r in (a0, a1, a2, a3, fwd_r, fwd_l):
            r.wait_send()
        for cp in out_copies:
            cp.wait()

    return pl.pallas_call(
        body,
        out_shape=jax.ShapeDtypeStruct((N_DEV * m_per, n_per), jnp.float32),
        in_specs=[
            pl.BlockSpec(memory_space=pltpu.MemorySpace.HBM),
            pl.BlockSpec(memory_space=pltpu.MemorySpace.HBM),
        ],
        out_specs=pl.BlockSpec(memory_space=pltpu.MemorySpace.HBM),
        scratch_shapes=[
            pltpu.VMEM((m_per, k), jnp.float32),
            pltpu.VMEM((k, n_per), jnp.float32),
            pltpu.VMEM((N_DEV * m_per, n_per), jnp.float32),
            pltpu.VMEM((m_per, k), jnp.float32),
            pltpu.VMEM((m_per, k), jnp.float32),
            pltpu.VMEM((m_per, k), jnp.float32),
            pltpu.SemaphoreType.DMA((6,)),
            pltpu.SemaphoreType.DMA((6,)),
            pltpu.SemaphoreType.DMA((2,)),
            pltpu.SemaphoreType.DMA((8,)),
        ],
        compiler_params=pltpu.CompilerParams(collective_id=0),
    )(x, w_mat)
